# baseline (speedup 1.0000x reference)
import functools

import jax
import jax.numpy as jnp
from jax.experimental import pallas as pl
from jax.experimental.pallas import tpu as pltpu


def _sage_kernel(a_ref, h_ref, wt_ref, o_ref, acc_ref, deg_ref, *,
                 n_k, k_blk, row_blk):
    i = pl.program_id(0)
    k = pl.program_id(1)

    @pl.when(k == 0)
    def _init():
        acc_ref[...] = jnp.zeros_like(acc_ref)
        deg_ref[...] = jnp.zeros_like(deg_ref)

    a = a_ref[:, 0, 0, :]
    hk = h_ref[pl.ds(k * k_blk, k_blk), :]
    acc_ref[...] += jnp.dot(a, hk, preferred_element_type=jnp.float32)
    deg_ref[...] += jnp.sum(a, axis=1, keepdims=True)

    @pl.when(k == n_k - 1)
    def _epilogue():
        hr = h_ref[pl.ds(i * row_blk, row_blk), :]
        agg = (acc_ref[...] + hr) / (deg_ref[...] + 1.0)
        z = jnp.dot(agg, wt_ref[...], preferred_element_type=jnp.float32)
        o_ref[...] = jnp.where(z >= 0.0, z, 0.01 * z)


def _pick_block(n, target):
    if n % target == 0:
        return target
    return n


def kernel(A, h, weight):
    n, d = h.shape
    row_blk = _pick_block(n, 2000)
    k_blk = _pick_block(n, 2500)
    n_k = n // k_blk
    grid = (n // row_blk, n_k)

    a4 = A.reshape(n, n_k, 1, k_blk)
    wt = weight.T

    out = pl.pallas_call(
        functools.partial(_sage_kernel, n_k=n_k, k_blk=k_blk, row_blk=row_blk),
        grid=grid,
        in_specs=[
            pl.BlockSpec((row_blk, 1, 1, k_blk), lambda i, k: (i, k, 0, 0)),
            pl.BlockSpec((n, d), lambda i, k: (0, 0)),
            pl.BlockSpec((d, d), lambda i, k: (0, 0)),
        ],
        out_specs=pl.BlockSpec((row_blk, d), lambda i, k: (i, 0)),
        out_shape=jax.ShapeDtypeStruct((n, d), jnp.float32),
        scratch_shapes=[
            pltpu.VMEM((row_blk, d), jnp.float32),
            pltpu.VMEM((row_blk, 1), jnp.float32),
        ],
        compiler_params=pltpu.CompilerParams(
            dimension_semantics=("parallel", "arbitrary"),
        ),
    )(a4, h, wt)
    return out


# speedup vs baseline: 11.6202x; 11.6202x over previous
import functools

import jax
import jax.numpy as jnp
from jax.experimental import pallas as pl
from jax.experimental.pallas import tpu as pltpu


def _sage_kernel(a_ref, h_ref, wt_ref, o_ref, *, row_blk):
    i = pl.program_id(0)
    a = a_ref[...]
    s = jax.lax.dot_general(
        a, h_ref[...], (((1,), (0,)), ((), ())),
        precision=jax.lax.Precision.DEFAULT,
        preferred_element_type=jnp.float32,
    )
    deg = jnp.sum(a, axis=1, keepdims=True)
    hr = h_ref[pl.ds(i * row_blk, row_blk), :]
    agg = (s + hr) / (deg + 1.0)
    z = jnp.dot(agg, wt_ref[...], preferred_element_type=jnp.float32)
    o_ref[...] = jnp.where(z >= 0.0, z, 0.01 * z)


def _pick_block(n, target):
    if n % target == 0:
        return target
    return n


def kernel(A, h, weight):
    n, d = h.shape
    row_blk = _pick_block(n, 400)
    wt = weight.T

    out = pl.pallas_call(
        functools.partial(_sage_kernel, row_blk=row_blk),
        grid=(n // row_blk,),
        in_specs=[
            pl.BlockSpec((row_blk, n), lambda i: (i, 0)),
            pl.BlockSpec((n, d), lambda i: (0, 0)),
            pl.BlockSpec((d, d), lambda i: (0, 0)),
        ],
        out_specs=pl.BlockSpec((row_blk, d), lambda i: (i, 0)),
        out_shape=jax.ShapeDtypeStruct((n, d), jnp.float32),
        compiler_params=pltpu.CompilerParams(
            dimension_semantics=("arbitrary",),
        ),
    )(A, h, wt)
    return out
